# speedup vs baseline: 1.5448x; 1.5448x over previous
import functools

import jax
import jax.numpy as jnp
from jax import lax
from jax.experimental import pallas as pl
from jax.experimental.pallas import tpu as pltpu
from jax.experimental.pallas import tpu_sc as plsc

VOCAB = 100000
D_MODEL = 4096
BATCH = 4
SEQ = 2048
N_TOK = BATCH * SEQ

NUM_WORKERS = 32
TOK_PER_W = N_TOK // NUM_WORKERS
CHUNK = 8
NCHUNKS = TOK_PER_W // CHUNK

MASK_MIN = float(jnp.finfo(jnp.float32).min)
MASK_BLK = 256


def _gather_sc(ids_flat, table):
    mesh = plsc.VectorSubcoreMesh(core_axis_name="c", subcore_axis_name="s")

    @functools.partial(
        pl.kernel,
        mesh=mesh,
        out_type=jax.ShapeDtypeStruct((N_TOK, D_MODEL), jnp.float32),
        scratch_types=[
            pltpu.VMEM((TOK_PER_W,), jnp.int32),
            pltpu.VMEM((CHUNK, D_MODEL), jnp.float32),
            pltpu.SemaphoreType.DMA,
        ],
    )
    def k(ids_hbm, table_hbm, out_hbm, idx_v, rows_v, sem):
        wid = lax.axis_index("s") * 2 + lax.axis_index("c")
        base = pl.multiple_of(wid * TOK_PER_W, TOK_PER_W)
        pltpu.sync_copy(ids_hbm.at[pl.ds(base, TOK_PER_W)], idx_v)

        def body(g, carry):
            off = pl.multiple_of(g * CHUNK, CHUNK)
            pltpu.async_copy(
                table_hbm.at[idx_v.at[pl.ds(off, CHUNK)]], rows_v, sem
            ).wait()
            pltpu.sync_copy(rows_v, out_hbm.at[pl.ds(base + off, CHUNK)])
            return carry

        lax.fori_loop(0, NCHUNKS, body, 0)

    return k(ids_flat, table)


def _mask_body(mask_ref, pos_ref):
    j = pl.program_id(1)
    row = lax.broadcasted_iota(jnp.int32, (MASK_BLK, SEQ), 0) + j * MASK_BLK
    col = lax.broadcasted_iota(jnp.int32, (MASK_BLK, SEQ), 1)
    mask_ref[0, 0] = jnp.where(col <= row, 0.0, MASK_MIN).astype(jnp.float32)
    pos_ref[...] = lax.broadcasted_iota(jnp.int32, (1, SEQ), 1)


def _mask_tc():
    return pl.pallas_call(
        _mask_body,
        grid=(BATCH, SEQ // MASK_BLK),
        out_shape=(
            jax.ShapeDtypeStruct((BATCH, 1, SEQ, SEQ), jnp.float32),
            jax.ShapeDtypeStruct((1, SEQ), jnp.int32),
        ),
        out_specs=(
            pl.BlockSpec((1, 1, MASK_BLK, SEQ), lambda b, j: (b, 0, j, 0)),
            pl.BlockSpec((1, SEQ), lambda b, j: (0, 0)),
        ),
    )()


def kernel(input_ids, embed_tokens):
    ids_flat = input_ids.reshape(-1).astype(jnp.int32)
    embeds = _gather_sc(ids_flat, embed_tokens)
    hidden = embeds.reshape(BATCH, SEQ, D_MODEL)
    combined_mask, position_ids = _mask_tc()
    return (hidden, combined_mask, position_ids)


# speedup vs baseline: 1.7218x; 1.1146x over previous
import functools

import jax
import jax.numpy as jnp
from jax import lax
from jax.experimental import pallas as pl
from jax.experimental.pallas import tpu as pltpu
from jax.experimental.pallas import tpu_sc as plsc

VOCAB = 100000
D_MODEL = 4096
BATCH = 4
SEQ = 2048
N_TOK = BATCH * SEQ

NUM_WORKERS = 32
TOK_PER_W = N_TOK // NUM_WORKERS
CHUNK = 8
NCHUNKS = TOK_PER_W // CHUNK

MASK_MIN = float(jnp.finfo(jnp.float32).min)
MASK_BLK = 256


def _gather_sc(ids_flat, table):
    mesh = plsc.VectorSubcoreMesh(core_axis_name="c", subcore_axis_name="s")

    @functools.partial(
        pl.kernel,
        mesh=mesh,
        out_type=jax.ShapeDtypeStruct((N_TOK, D_MODEL), jnp.float32),
        scratch_types=[
            pltpu.VMEM((TOK_PER_W,), jnp.int32),
            pltpu.VMEM((CHUNK, D_MODEL), jnp.float32),
            pltpu.VMEM((CHUNK, D_MODEL), jnp.float32),
            pltpu.VMEM((CHUNK, D_MODEL), jnp.float32),
            pltpu.SemaphoreType.DMA,
            pltpu.SemaphoreType.DMA,
            pltpu.SemaphoreType.DMA,
            pltpu.SemaphoreType.DMA,
            pltpu.SemaphoreType.DMA,
            pltpu.SemaphoreType.DMA,
        ],
    )
    def k(ids_hbm, table_hbm, out_hbm, idx_v, r0, r1, r2, g0, g1, g2, o0, o1, o2):
        rows = (r0, r1, r2)
        gsems = (g0, g1, g2)
        osems = (o0, o1, o2)
        wid = lax.axis_index("s") * 2 + lax.axis_index("c")
        base = pl.multiple_of(wid * TOK_PER_W, TOK_PER_W)
        pltpu.sync_copy(ids_hbm.at[pl.ds(base, TOK_PER_W)], idx_v)

        def idx_slice(c):
            return idx_v.at[pl.ds(pl.multiple_of(c * CHUNK, CHUNK), CHUNK)]

        def out_slice(c):
            return out_hbm.at[pl.ds(base + pl.multiple_of(c * CHUNK, CHUNK), CHUNK)]

        def fire_gather(c, b):
            pltpu.async_copy(table_hbm.at[idx_slice(c)], rows[b], gsems[b])

        def wait_gather(c, b):
            pltpu.make_async_copy(table_hbm.at[idx_slice(c)], rows[b], gsems[b]).wait()

        def fire_out(c, b):
            pltpu.async_copy(rows[b], out_slice(c), osems[b])

        def wait_out(c, b):
            pltpu.make_async_copy(rows[b], out_slice(c), osems[b]).wait()

        fire_gather(0, 0)
        fire_gather(1, 1)

        def body(i, carry):
            for bb in range(3):
                c = i * 3 + bb
                wait_gather(c, bb)
                fire_out(c, bb)
                nb = (bb + 2) % 3
                if bb == 0:
                    @pl.when(i > 0)
                    def _():
                        wait_out(c - 1, nb)
                else:
                    wait_out(c - 1, nb)
                fire_gather(c + 2, nb)
            return carry

        lax.fori_loop(0, NCHUNKS // 3, body, 0)

        for c, bb in ((30, 0), (31, 1)):
            wait_gather(c, bb)
            fire_out(c, bb)
        wait_out(29, 2)
        wait_out(30, 0)
        wait_out(31, 1)

    return k(ids_flat, table)


def _mask_body(mask_ref, pos_ref):
    j = pl.program_id(1)
    row = lax.broadcasted_iota(jnp.int32, (MASK_BLK, SEQ), 0) + j * MASK_BLK
    col = lax.broadcasted_iota(jnp.int32, (MASK_BLK, SEQ), 1)
    mask_ref[0, 0] = jnp.where(col <= row, 0.0, MASK_MIN).astype(jnp.float32)
    pos_ref[...] = lax.broadcasted_iota(jnp.int32, (1, SEQ), 1)


def _mask_tc():
    return pl.pallas_call(
        _mask_body,
        grid=(BATCH, SEQ // MASK_BLK),
        out_shape=(
            jax.ShapeDtypeStruct((BATCH, 1, SEQ, SEQ), jnp.float32),
            jax.ShapeDtypeStruct((1, SEQ), jnp.int32),
        ),
        out_specs=(
            pl.BlockSpec((1, 1, MASK_BLK, SEQ), lambda b, j: (b, 0, j, 0)),
            pl.BlockSpec((1, SEQ), lambda b, j: (0, 0)),
        ),
    )()


def kernel(input_ids, embed_tokens):
    ids_flat = input_ids.reshape(-1).astype(jnp.int32)
    embeds = _gather_sc(ids_flat, embed_tokens)
    hidden = embeds.reshape(BATCH, SEQ, D_MODEL)
    combined_mask, position_ids = _mask_tc()
    return (hidden, combined_mask, position_ids)
